# Initial kernel scaffold; baseline (speedup 1.0000x reference)
import jax
import jax.numpy as jnp
from jax.experimental import pallas as pl


def kernel(x, C, A):
    raise NotImplementedError("write your pallas kernel here")


# speedup vs baseline: 1.0060x; 1.0060x over previous
import functools

import jax
import jax.numpy as jnp
from jax.experimental import pallas as pl
from jax.experimental.pallas import tpu as pltpu


def _ceil_to(x, m):
    return ((x + m - 1) // m) * m


def _largest_divisor(n, cap):
    for t in range(min(n, cap), 0, -1):
        if n % t == 0:
            return t
    return 1


def _proj_kernel(y_ref, c_ref, u_ref, abspart_ref):
    T = y_ref.shape[0]
    bp = y_ref.shape[1]
    yblk = y_ref[...]
    yflat = yblk.reshape(T * bp, y_ref.shape[2])
    u = jnp.dot(yflat, c_ref[...], preferred_element_type=jnp.float32)
    u_ref[...] = u.reshape(T, bp, c_ref.shape[1])
    abspart_ref[0] = jnp.sum(jnp.abs(yflat), keepdims=True)


def _powers_kernel(a_ref, c_ref, u_ref, g_ref, mtm_ref, rhs_ref):
    s = u_ref.shape[0]
    h = a_ref.shape[0]
    a = a_ref[...]
    c = c_ref[...]
    S = jax.lax.dot_general(c, c, dimension_numbers=(((0,), (0,)), ((), ())),
                            preferred_element_type=jnp.float32)
    rows = jax.lax.broadcasted_iota(jnp.int32, (h, h), 0)
    cols = jax.lax.broadcasted_iota(jnp.int32, (h, h), 1)
    oc = (rows == cols).astype(jnp.float32)
    mtm = jnp.zeros((h, h), jnp.float32)
    rhs = jnp.zeros((u_ref.shape[1], h), jnp.float32)
    for k in range(s):
        g_ref[:, k * h:(k + 1) * h] = oc
        sg = jnp.dot(S, oc, preferred_element_type=jnp.float32)
        mtm = mtm + jax.lax.dot_general(
            oc, sg, dimension_numbers=(((0,), (0,)), ((), ())),
            preferred_element_type=jnp.float32)
        rhs = rhs + jnp.dot(u_ref[k], oc, preferred_element_type=jnp.float32)
        oc = jnp.dot(oc, a, preferred_element_type=jnp.float32)
    mtm_ref[...] = mtm
    rhs_ref[...] = rhs


def _emit_kernel(y_ref, c_ref, g_ref, x0t_ref, invlam_ref,
                 o_ref, yhat_ref, w_ref, sqpart_ref, xs_ref):
    T = y_ref.shape[0]
    bp = y_ref.shape[1]
    obs = y_ref.shape[2]
    h = x0t_ref.shape[1]
    c = c_ref[...]
    inv_lam = invlam_ref[0]
    ores = jnp.dot(c, g_ref[...], preferred_element_type=jnp.float32)
    x0t = x0t_ref[...]
    for j in range(T):
        o_ref[j] = ores[:, j * h:(j + 1) * h]
        xs_ref[j * bp:(j + 1) * bp, :] = jax.lax.dot_general(
            x0t, g_ref[:, j * h:(j + 1) * h],
            dimension_numbers=(((1,), (1,)), ((), ())),
            preferred_element_type=jnp.float32)
    yh = jax.lax.dot_general(
        xs_ref[...], c, dimension_numbers=(((1,), (1,)), ((), ())),
        preferred_element_type=jnp.float32)
    yflat = y_ref[...].reshape(T * bp, obs)
    w = (yflat - yh) * inv_lam
    yhat_ref[...] = yh.reshape(T, bp, obs)
    w_ref[...] = w.reshape(T, bp, obs)
    sqpart_ref[0] = jnp.sum(w * w, keepdims=True)


@jax.jit
def _forward(x, C, A):
    b, s, c, h, w = x.shape
    obs = c * h * w
    hidden = C.shape[1]
    M = float(obs)
    prediction_alpha = 1.0

    obs_p = _ceil_to(obs, 128)
    hid_p = _ceil_to(hidden, 128)
    b_p = _ceil_to(b, 8)
    T1 = _largest_divisor(s, 8)
    T3 = _largest_divisor(s, 4)

    y = x.reshape(b, s, obs).astype(jnp.float32)
    logdet = jnp.zeros((b * s,), jnp.float32)
    Y = y.reshape(b, s * obs)

    y_sbo = jnp.transpose(y, (1, 0, 2))
    if (b_p, obs_p) != (b, obs):
        y_pad = jnp.zeros((s, b_p, obs_p), jnp.float32).at[:, :b, :obs].set(y_sbo)
    else:
        y_pad = y_sbo
    if (obs_p, hid_p) != (obs, hidden):
        C_pad = jnp.zeros((obs_p, hid_p), jnp.float32).at[:obs, :hidden].set(C)
    else:
        C_pad = C
    if hid_p != hidden:
        A_pad = jnp.zeros((hid_p, hid_p), jnp.float32).at[:hidden, :hidden].set(A)
    else:
        A_pad = A

    n1 = s // T1
    u, abspart = pl.pallas_call(
        _proj_kernel,
        out_shape=(
            jax.ShapeDtypeStruct((s, b_p, hid_p), jnp.float32),
            jax.ShapeDtypeStruct((n1, 1, 1), jnp.float32),
        ),
        grid_spec=pltpu.PrefetchScalarGridSpec(
            num_scalar_prefetch=0,
            grid=(n1,),
            in_specs=[
                pl.BlockSpec((T1, b_p, obs_p), lambda i: (i, 0, 0)),
                pl.BlockSpec((obs_p, hid_p), lambda i: (0, 0)),
            ],
            out_specs=[
                pl.BlockSpec((T1, b_p, hid_p), lambda i: (i, 0, 0)),
                pl.BlockSpec((1, 1, 1), lambda i: (i, 0, 0)),
            ],
        ),
        compiler_params=pltpu.CompilerParams(
            dimension_semantics=("parallel",),
            vmem_limit_bytes=48 * 1024 * 1024),
    )(y_pad, C_pad)

    G, MtM_pad, rhs_pad = pl.pallas_call(
        _powers_kernel,
        out_shape=(
            jax.ShapeDtypeStruct((hid_p, s * hid_p), jnp.float32),
            jax.ShapeDtypeStruct((hid_p, hid_p), jnp.float32),
            jax.ShapeDtypeStruct((b_p, hid_p), jnp.float32),
        ),
        grid_spec=pltpu.PrefetchScalarGridSpec(
            num_scalar_prefetch=0,
            grid=(1,),
            in_specs=[
                pl.BlockSpec((hid_p, hid_p), lambda i: (0, 0)),
                pl.BlockSpec((obs_p, hid_p), lambda i: (0, 0)),
                pl.BlockSpec((s, b_p, hid_p), lambda i: (0, 0, 0)),
            ],
            out_specs=[
                pl.BlockSpec((hid_p, s * hid_p), lambda i: (0, 0)),
                pl.BlockSpec((hid_p, hid_p), lambda i: (0, 0)),
                pl.BlockSpec((b_p, hid_p), lambda i: (0, 0)),
            ],
        ),
        compiler_params=pltpu.CompilerParams(
            dimension_semantics=("arbitrary",),
            vmem_limit_bytes=48 * 1024 * 1024),
    )(A_pad, C_pad, u)

    abssum = jnp.sum(abspart)
    scaling_lambda = abssum / (b * s * obs)

    MtM = MtM_pad[:hidden, :hidden]
    rhs = rhs_pad[:b, :hidden]
    L = jnp.linalg.cholesky(MtM)
    z = jax.scipy.linalg.solve_triangular(L, rhs.T, lower=True)
    x0 = jax.scipy.linalg.solve_triangular(L.T, z, lower=False)

    if (b_p, hid_p) != (b, hidden):
        x0t_pad = jnp.zeros((b_p, hid_p), jnp.float32).at[:b, :hidden].set(x0.T)
    else:
        x0t_pad = x0.T
    inv_lambda = (1.0 / scaling_lambda).reshape(1).astype(jnp.float32)

    n3 = s // T3
    O_pad, Yhat_pad, W_pad, sqpart = pl.pallas_call(
        _emit_kernel,
        out_shape=(
            jax.ShapeDtypeStruct((s, obs_p, hid_p), jnp.float32),
            jax.ShapeDtypeStruct((s, b_p, obs_p), jnp.float32),
            jax.ShapeDtypeStruct((s, b_p, obs_p), jnp.float32),
            jax.ShapeDtypeStruct((n3, 1, 1), jnp.float32),
        ),
        grid_spec=pltpu.PrefetchScalarGridSpec(
            num_scalar_prefetch=0,
            grid=(n3,),
            in_specs=[
                pl.BlockSpec((T3, b_p, obs_p), lambda i: (i, 0, 0)),
                pl.BlockSpec((obs_p, hid_p), lambda i: (0, 0)),
                pl.BlockSpec((hid_p, T3 * hid_p), lambda i: (0, i)),
                pl.BlockSpec((b_p, hid_p), lambda i: (0, 0)),
                pl.BlockSpec(memory_space=pltpu.MemorySpace.SMEM),
            ],
            out_specs=[
                pl.BlockSpec((T3, obs_p, hid_p), lambda i: (i, 0, 0)),
                pl.BlockSpec((T3, b_p, obs_p), lambda i: (i, 0, 0)),
                pl.BlockSpec((T3, b_p, obs_p), lambda i: (i, 0, 0)),
                pl.BlockSpec((1, 1, 1), lambda i: (i, 0, 0)),
            ],
            scratch_shapes=[
                pltpu.VMEM((T3 * b_p, hid_p), jnp.float32),
            ],
        ),
        compiler_params=pltpu.CompilerParams(
            dimension_semantics=("parallel",),
            vmem_limit_bytes=48 * 1024 * 1024),
    )(y_pad, C_pad, G, x0t_pad, inv_lambda)

    prediction_error = jnp.sum(sqpart) / (s * obs * b)
    log_likelihood = jnp.mean(logdet / M) - jnp.log(scaling_lambda)
    loss = -log_likelihood + prediction_alpha * prediction_error

    O = O_pad[:, :obs, :hidden].reshape(s * obs, hidden)
    Yhat = jnp.transpose(Yhat_pad[:, :b, :obs], (0, 2, 1)).reshape(s * obs, b)
    W = jnp.transpose(W_pad[:, :b, :obs], (0, 2, 1)).reshape(s * obs, b)
    return loss, (W, Y, Yhat, y, x0, logdet / M, prediction_error, O,
                  scaling_lambda)


def kernel(x, C, A):
    return _forward(x, C, A)
